# baseline (speedup 1.0000x reference)
import functools
import numpy as np
import jax
import jax.numpy as jnp
from jax import lax
from jax.experimental import pallas as pl
from jax.experimental.pallas import tpu as pltpu
from jax.experimental.pallas import tpu_sc as plsc

_SPATIAL = [(80, 80), (40, 40), (20, 20)]
_B, _Q, _D, _H, _L, _P = 8, 300, 256, 8, 3, 4
_HD = _D // _H
_LEN_V = sum(h * w for h, w in _SPATIAL)
_BQ = _B * _Q
_ROWS = _BQ * _H
_E = _L * _P * 4

_starts = np.concatenate([[0], np.cumsum([h * w for h, w in _SPATIAL])])
_k96 = np.arange(96)
_h_lane = _k96 // 12
_lp_lane = _k96 % 12
_l_lane = _lp_lane // 4
_SxA = np.array([_SPATIAL[l][1] for l in _l_lane], np.float32)[None]
_SyA = np.array([_SPATIAL[l][0] for l in _l_lane], np.float32)[None]
_strideA = (_SxA * _H).astype(np.float32)
_baseHA = (_starts[_l_lane] * _H + _h_lane).astype(np.float32)[None]
_G_seg = np.kron(np.eye(_H, dtype=np.float32), np.ones((12, 12), np.float32))

_chan_perm = np.zeros(_D, np.int32)
for _hh in range(_H):
    for _c2 in range(16):
        _chan_perm[_hh * 32 + _c2] = _hh * 32 + 2 * _c2
        _chan_perm[_hh * 32 + 16 + _c2] = _hh * 32 + 2 * _c2 + 1

_Ecat = np.zeros((12, 512), np.float32)
for _comp in range(4):
    for _kk in range(96):
        _Ecat[_l_lane[_kk] * 4 + _comp, _comp * 128 + _kk] = 1.0

_Pc = np.zeros((4, 96, 384), np.float32)
for _c in range(4):
    for _kk in range(96):
        _Pc[_c, _kk, _h_lane[_kk] * 48 + _c * 12 + _lp_lane[_kk]] = 1.0
_Px0 = _Pc[0] + _Pc[2]
_Px1 = _Pc[1] + _Pc[3]
_Py0 = _Pc[0] + _Pc[1]
_Py1 = _Pc[2] + _Pc[3]
_m384 = np.arange(384)
_hp = _m384 // 48
_lpp = _m384 % 12
_lP = _lpp // 4
_WlP = np.array([_SPATIAL[l][1] for l in _lP], np.float32)
_strideP = (_WlP * _H).astype(np.float32)[None]
_basePm = (_starts[_lP] * _H + _hp).astype(np.float32)[None]

_VROWS = _B * _LEN_V
_VBLK = 1600


def _vproj_body(x_ref, w_ref, b_ref, o_ref):
    o_ref[...] = (
        jnp.dot(x_ref[...], w_ref[...], preferred_element_type=jnp.float32)
        + b_ref[...]
    ).astype(jnp.bfloat16)


def _vproj(x, w, b):
    return pl.pallas_call(
        _vproj_body,
        grid=(_VROWS // _VBLK,),
        in_specs=[
            pl.BlockSpec((_VBLK, _D), lambda i: (i, 0)),
            pl.BlockSpec((_D, _D), lambda i: (0, 0)),
            pl.BlockSpec((1, _D), lambda i: (0, 0)),
        ],
        out_specs=pl.BlockSpec((_VBLK, _D), lambda i: (i, 0)),
        out_shape=jax.ShapeDtypeStruct((_VROWS, _D), jnp.bfloat16),
    )(x, w, b)


def _split_dot(a, p):
    f32 = jnp.float32
    ahi = a.astype(jnp.bfloat16).astype(f32)
    alo = a - ahi
    return (jnp.dot(ahi, p, preferred_element_type=f32)
            + jnp.dot(alo, p, preferred_element_type=f32))


def _meta_body(q_ref, wox_ref, woy_ref, wat_ref, box_ref, boy_ref, bat_ref,
               g_ref, ecat_ref, rp_ref, sx_ref, sy_ref,
               px0_ref, px1_ref, py0_ref, py1_ref,
               p0_ref, p1_ref, p2_ref, p3_ref,
               stridep_ref, basep_ref, idx_ref, w_ref):
    q = q_ref[...]
    f32 = jnp.float32
    offx = jnp.dot(q, wox_ref[...], preferred_element_type=f32) + box_ref[...]
    offy = jnp.dot(q, woy_ref[...], preferred_element_type=f32) + boy_ref[...]
    logit = jnp.dot(q, wat_ref[...], preferred_element_type=f32) + bat_ref[...]
    m = jnp.max(logit, axis=1, keepdims=True)
    e = jnp.exp(logit - m)
    denom = jnp.dot(e, g_ref[...], preferred_element_type=f32)
    aw = e / denom

    rpe = _split_dot(rp_ref[...], ecat_ref[...])
    rpx = rpe[:, 0:96]
    rpy = rpe[:, 128:224]
    rpw = rpe[:, 256:352]
    rph = rpe[:, 384:480]

    sx = sx_ref[...]
    sy = sy_ref[...]
    px = (rpx + offx * rpw * 0.125) * sx - 0.5
    py = (rpy + offy * rph * 0.125) * sy - 0.5
    x0 = jnp.floor(px)
    y0 = jnp.floor(py)
    fx = px - x0
    fy = py - y0
    vx0 = ((x0 >= 0) & (x0 < sx)).astype(f32)
    vx1 = ((x0 >= -1) & (x0 < sx - 1)).astype(f32)
    vy0 = ((y0 >= 0) & (y0 < sy)).astype(f32)
    vy1 = ((y0 >= -1) & (y0 < sy - 1)).astype(f32)
    cx0 = jnp.clip(x0, 0, sx - 1)
    cx1 = jnp.clip(x0 + 1, 0, sx - 1)
    cy0 = jnp.clip(y0, 0, sy - 1)
    cy1 = jnp.clip(y0 + 1, 0, sy - 1)

    w00 = (1 - fx) * (1 - fy) * vx0 * vy0 * aw
    w01 = fx * (1 - fy) * vx1 * vy0 * aw
    w10 = (1 - fx) * fy * vx0 * vy1 * aw
    w11 = fx * fy * vx1 * vy1 * aw

    cxp = (jnp.dot(cx0, px0_ref[...], preferred_element_type=f32)
           + jnp.dot(cx1, px1_ref[...], preferred_element_type=f32))
    cyp = (jnp.dot(cy0, py0_ref[...], preferred_element_type=f32)
           + jnp.dot(cy1, py1_ref[...], preferred_element_type=f32))
    r = lax.broadcasted_iota(jnp.int32, (_BQ, 384), 0)
    bH = ((r // _Q) * (_LEN_V * _H)).astype(f32)
    ip = bH + basep_ref[...] + cyp * stridep_ref[...] + cxp * float(_H)

    wp = (_split_dot(w00, p0_ref[...]) + _split_dot(w01, p1_ref[...])
          + _split_dot(w10, p2_ref[...]) + _split_dot(w11, p3_ref[...]))
    idx_ref[...] = jnp.floor(ip + 0.5).astype(jnp.int32)
    w_ref[...] = wp


def _meta(q2, wox, woy, wat, box, boy, bat, g, ecat, rp12, sx, sy):
    return pl.pallas_call(
        _meta_body,
        out_shape=[
            jax.ShapeDtypeStruct((_BQ, 384), jnp.int32),
            jax.ShapeDtypeStruct((_BQ, 384), jnp.float32),
        ],
    )(q2, wox, woy, wat, box, boy, bat, g, ecat, rp12, sx, sy,
      jnp.asarray(_Px0), jnp.asarray(_Px1), jnp.asarray(_Py0),
      jnp.asarray(_Py1), jnp.asarray(_Pc[0]), jnp.asarray(_Pc[1]),
      jnp.asarray(_Pc[2]), jnp.asarray(_Pc[3]), jnp.asarray(_strideP),
      jnp.asarray(_basePm))


_NC, _NS = 2, 16
_NW = _NC * _NS
_BQPW = _BQ // _NW
_K = 5
_NCHUNK = _BQPW // _K
_RPC = _K * _H


def _sc_body(table_ref, idx_ref, w_ref, out_ref, idx_v, w_v, g_v, o_v, sem):
    wid = lax.axis_index("s") * _NC + lax.axis_index("c")
    bq0 = wid * _BQPW

    def chunk(ci, carry):
        bqs = bq0 + ci * _K
        pltpu.sync_copy(idx_ref.at[pl.ds(bqs, _K)], idx_v)
        pltpu.sync_copy(w_ref.at[pl.ds(bqs * 384, _K * 384)],
                        w_v.at[pl.ds(16, _K * 384)])
        copies = []
        for r in range(_RPC):
            copies.append(
                pltpu.async_copy(
                    table_ref.at[idx_v.at[r // _H, pl.ds((r % _H) * _E, _E)]],
                    g_v.at[r], sem
                )
            )
        for r in range(_RPC):
            copies[r].wait()
            off = r * _E
            a0 = jnp.zeros((16,), jnp.float32)
            a1 = jnp.zeros((16,), jnp.float32)
            for g in range(3):
                wv = w_v[pl.ds(16 + off + g * 16, 16)]
                for t in range(16):
                    j = g * 16 + t
                    wj = jnp.full((16,), wv[t])
                    lo, hi = plsc.unpack(
                        g_v[r, j, pl.ds(0, 32)],
                        format=plsc.PackFormat.INTERLEAVED,
                    )
                    a0 = a0 + wj * lo.astype(jnp.float32)
                    a1 = a1 + wj * hi.astype(jnp.float32)
            o_v[r, pl.ds(0, 16)] = a0
            o_v[r, pl.ds(16, 16)] = a1
        pltpu.sync_copy(o_v, out_ref.at[pl.ds(bqs * _H, _RPC)])
        return carry

    lax.fori_loop(0, _NCHUNK, chunk, 0)


def _sc_gather(table, idxs, ws):
    mesh = plsc.VectorSubcoreMesh(
        core_axis_name="c", subcore_axis_name="s", num_cores=_NC,
        num_subcores=_NS,
    )
    f = pl.kernel(
        _sc_body,
        out_type=jax.ShapeDtypeStruct((_ROWS, _HD), jnp.float32),
        mesh=mesh,
        compiler_params=pltpu.CompilerParams(
            needs_layout_passes=False, use_tc_tiling_on_sc=False
        ),
        scratch_types=[
            pltpu.VMEM((_K, 384), jnp.int32),
            pltpu.VMEM((16 + _K * 384,), jnp.float32),
            pltpu.VMEM((_RPC, _E, _HD), jnp.bfloat16),
            pltpu.VMEM((_RPC, _HD), jnp.float32),
            pltpu.SemaphoreType.DMA,
        ],
    )
    return f(table, idxs, ws)


def _outproj_body(x_ref, w_ref, b_ref, o_ref):
    o_ref[...] = (
        jnp.dot(x_ref[...], w_ref[...], preferred_element_type=jnp.float32)
        + b_ref[...]
    )


def _outproj(x, w, b):
    return pl.pallas_call(
        _outproj_body,
        out_shape=jax.ShapeDtypeStruct((_BQ, _D), jnp.float32),
    )(x, w, b)


def kernel(query, reference_points, value, W_val, b_val, W_off, b_off,
           W_attn, b_attn, W_out, b_out):
    table = _vproj(value.reshape(_VROWS, _D), W_val, b_val[None, :])

    idxs, ws = _meta(
        query.reshape(_BQ, _D),
        W_off[:, 0::2], W_off[:, 1::2], W_attn,
        b_off[None, 0::2], b_off[None, 1::2], b_attn[None, :],
        jnp.asarray(_G_seg), jnp.asarray(_Ecat),
        reference_points.reshape(_BQ, 12),
        jnp.asarray(_SxA), jnp.asarray(_SyA),
    )

    sc_out = _sc_gather(table.reshape(_VROWS * _H, _HD),
                        idxs, ws.reshape(_BQ * 384))

    out = _outproj(sc_out.reshape(_BQ, _D), W_out[jnp.asarray(_chan_perm)],
                   b_out[None, :])
    return out.reshape(_B, _Q, _D)


# speedup vs baseline: 1.1004x; 1.1004x over previous
import functools
import numpy as np
import jax
import jax.numpy as jnp
from jax import lax
from jax.experimental import pallas as pl
from jax.experimental.pallas import tpu as pltpu
from jax.experimental.pallas import tpu_sc as plsc

_SPATIAL = [(80, 80), (40, 40), (20, 20)]
_B, _Q, _D, _H, _L, _P = 8, 300, 256, 8, 3, 4
_HD = _D // _H
_LEN_V = sum(h * w for h, w in _SPATIAL)
_BQ = _B * _Q
_ROWS = _BQ * _H
_E = _L * _P * 4

_starts = np.concatenate([[0], np.cumsum([h * w for h, w in _SPATIAL])])
_k96 = np.arange(96)
_h_lane = _k96 // 12
_lp_lane = _k96 % 12
_l_lane = _lp_lane // 4
_SxA = np.array([_SPATIAL[l][1] for l in _l_lane], np.float32)[None]
_SyA = np.array([_SPATIAL[l][0] for l in _l_lane], np.float32)[None]
_strideA = (_SxA * _H).astype(np.float32)
_baseHA = (_starts[_l_lane] * _H + _h_lane).astype(np.float32)[None]
_G_seg = np.kron(np.eye(_H, dtype=np.float32), np.ones((12, 12), np.float32))

_Ecat = np.zeros((12, 512), np.float32)
for _comp in range(4):
    for _kk in range(96):
        _Ecat[_l_lane[_kk] * 4 + _comp, _comp * 128 + _kk] = 1.0

_Pc = np.zeros((4, 96, 384), np.float32)
for _c in range(4):
    for _kk in range(96):
        _Pc[_c, _kk, _h_lane[_kk] * 48 + _c * 12 + _lp_lane[_kk]] = 1.0
_Px0 = _Pc[0] + _Pc[2]
_Px1 = _Pc[1] + _Pc[3]
_Py0 = _Pc[0] + _Pc[1]
_Py1 = _Pc[2] + _Pc[3]
_m384 = np.arange(384)
_hp = _m384 // 48
_lpp = _m384 % 12
_lP = _lpp // 4
_WlP = np.array([_SPATIAL[l][1] for l in _lP], np.float32)
_strideP = (_WlP * _H).astype(np.float32)[None]
_basePm = (_starts[_lP] * _H + _hp).astype(np.float32)[None]

_VROWS = _B * _LEN_V
_VBLK = 1600


def _vproj_body(x_ref, w_ref, b_ref, o_ref):
    o_ref[...] = (
        jnp.dot(x_ref[...], w_ref[...], preferred_element_type=jnp.float32)
        + b_ref[...]
    )


def _vproj(x, w, b):
    return pl.pallas_call(
        _vproj_body,
        grid=(_VROWS // _VBLK,),
        in_specs=[
            pl.BlockSpec((_VBLK, _D), lambda i: (i, 0)),
            pl.BlockSpec((_D, _D), lambda i: (0, 0)),
            pl.BlockSpec((1, _D), lambda i: (0, 0)),
        ],
        out_specs=pl.BlockSpec((_VBLK, _D), lambda i: (i, 0)),
        out_shape=jax.ShapeDtypeStruct((_VROWS, _D), jnp.float32),
    )(x, w, b)


def _split_dot(a, p):
    f32 = jnp.float32
    ahi = a.astype(jnp.bfloat16).astype(f32)
    alo = a - ahi
    return (jnp.dot(ahi, p, preferred_element_type=f32)
            + jnp.dot(alo, p, preferred_element_type=f32))


def _meta_body(q_ref, wox_ref, woy_ref, wat_ref, box_ref, boy_ref, bat_ref,
               g_ref, ecat_ref, rp_ref, sx_ref, sy_ref,
               px0_ref, px1_ref, py0_ref, py1_ref,
               p0_ref, p1_ref, p2_ref, p3_ref,
               stridep_ref, basep_ref, idx_ref, w_ref):
    q = q_ref[...]
    f32 = jnp.float32
    offx = jnp.dot(q, wox_ref[...], preferred_element_type=f32) + box_ref[...]
    offy = jnp.dot(q, woy_ref[...], preferred_element_type=f32) + boy_ref[...]
    logit = jnp.dot(q, wat_ref[...], preferred_element_type=f32) + bat_ref[...]
    m = jnp.max(logit, axis=1, keepdims=True)
    e = jnp.exp(logit - m)
    denom = jnp.dot(e, g_ref[...], preferred_element_type=f32)
    aw = e / denom

    rpe = _split_dot(rp_ref[...], ecat_ref[...])
    rpx = rpe[:, 0:96]
    rpy = rpe[:, 128:224]
    rpw = rpe[:, 256:352]
    rph = rpe[:, 384:480]

    sx = sx_ref[...]
    sy = sy_ref[...]
    px = (rpx + offx * rpw * 0.125) * sx - 0.5
    py = (rpy + offy * rph * 0.125) * sy - 0.5
    x0 = jnp.floor(px)
    y0 = jnp.floor(py)
    fx = px - x0
    fy = py - y0
    vx0 = ((x0 >= 0) & (x0 < sx)).astype(f32)
    vx1 = ((x0 >= -1) & (x0 < sx - 1)).astype(f32)
    vy0 = ((y0 >= 0) & (y0 < sy)).astype(f32)
    vy1 = ((y0 >= -1) & (y0 < sy - 1)).astype(f32)
    cx0 = jnp.clip(x0, 0, sx - 1)
    cx1 = jnp.clip(x0 + 1, 0, sx - 1)
    cy0 = jnp.clip(y0, 0, sy - 1)
    cy1 = jnp.clip(y0 + 1, 0, sy - 1)

    w00 = (1 - fx) * (1 - fy) * vx0 * vy0 * aw
    w01 = fx * (1 - fy) * vx1 * vy0 * aw
    w10 = (1 - fx) * fy * vx0 * vy1 * aw
    w11 = fx * fy * vx1 * vy1 * aw

    cxp = (jnp.dot(cx0, px0_ref[...], preferred_element_type=f32)
           + jnp.dot(cx1, px1_ref[...], preferred_element_type=f32))
    cyp = (jnp.dot(cy0, py0_ref[...], preferred_element_type=f32)
           + jnp.dot(cy1, py1_ref[...], preferred_element_type=f32))
    r = lax.broadcasted_iota(jnp.int32, (_BQ, 384), 0)
    bH = ((r // _Q) * (_LEN_V * _H)).astype(f32)
    ip = bH + basep_ref[...] + cyp * stridep_ref[...] + cxp * float(_H)

    wp = (_split_dot(w00, p0_ref[...]) + _split_dot(w01, p1_ref[...])
          + _split_dot(w10, p2_ref[...]) + _split_dot(w11, p3_ref[...]))
    idx_ref[...] = jnp.floor(ip + 0.5).astype(jnp.int32)
    w_ref[...] = wp


def _meta(q2, wox, woy, wat, box, boy, bat, g, ecat, rp12, sx, sy):
    return pl.pallas_call(
        _meta_body,
        out_shape=[
            jax.ShapeDtypeStruct((_BQ, 384), jnp.int32),
            jax.ShapeDtypeStruct((_BQ, 384), jnp.float32),
        ],
    )(q2, wox, woy, wat, box, boy, bat, g, ecat, rp12, sx, sy,
      jnp.asarray(_Px0), jnp.asarray(_Px1), jnp.asarray(_Py0),
      jnp.asarray(_Py1), jnp.asarray(_Pc[0]), jnp.asarray(_Pc[1]),
      jnp.asarray(_Pc[2]), jnp.asarray(_Pc[3]), jnp.asarray(_strideP),
      jnp.asarray(_basePm))


_NC, _NS = 2, 16
_NW = _NC * _NS
_BQPW = _BQ // _NW
_K = 5
_NCHUNK = _BQPW // _K
_RPC = _K * _H


def _sc_body(table_ref, idx_ref, w_ref, out_ref, idx_v, w_v, g_v, o_v, sem):
    wid = lax.axis_index("s") * _NC + lax.axis_index("c")
    bq0 = wid * _BQPW

    def chunk(ci, carry):
        bqs = bq0 + ci * _K
        pltpu.sync_copy(idx_ref.at[pl.ds(bqs, _K)], idx_v)
        pltpu.sync_copy(w_ref.at[pl.ds(bqs * 384, _K * 384)],
                        w_v.at[pl.ds(16, _K * 384)])
        copies = []
        for r in range(_RPC):
            copies.append(
                pltpu.async_copy(
                    table_ref.at[idx_v.at[r // _H, pl.ds((r % _H) * _E, _E)]],
                    g_v.at[r], sem
                )
            )
        for r in range(_RPC):
            copies[r].wait()
            off = r * _E
            a0 = jnp.zeros((16,), jnp.float32)
            a1 = jnp.zeros((16,), jnp.float32)
            for g in range(3):
                wv = w_v[pl.ds(16 + off + g * 16, 16)]
                for t in range(16):
                    j = g * 16 + t
                    wj = jnp.full((16,), wv[t])
                    a0 = a0 + wj * g_v[r, j, pl.ds(0, 16)]
                    a1 = a1 + wj * g_v[r, j, pl.ds(16, 16)]
            o_v[r, pl.ds(0, 16)] = a0
            o_v[r, pl.ds(16, 16)] = a1
        pltpu.sync_copy(o_v, out_ref.at[pl.ds(bqs * _H, _RPC)])
        return carry

    lax.fori_loop(0, _NCHUNK, chunk, 0)


def _sc_gather(table, idxs, ws):
    mesh = plsc.VectorSubcoreMesh(
        core_axis_name="c", subcore_axis_name="s", num_cores=_NC,
        num_subcores=_NS,
    )
    f = pl.kernel(
        _sc_body,
        out_type=jax.ShapeDtypeStruct((_ROWS, _HD), jnp.float32),
        mesh=mesh,
        compiler_params=pltpu.CompilerParams(
            needs_layout_passes=False, use_tc_tiling_on_sc=False
        ),
        scratch_types=[
            pltpu.VMEM((_K, 384), jnp.int32),
            pltpu.VMEM((16 + _K * 384,), jnp.float32),
            pltpu.VMEM((_RPC, _E, _HD), jnp.float32),
            pltpu.VMEM((_RPC, _HD), jnp.float32),
            pltpu.SemaphoreType.DMA,
        ],
    )
    return f(table, idxs, ws)


def _outproj_body(x_ref, w_ref, b_ref, o_ref):
    o_ref[...] = (
        jnp.dot(x_ref[...], w_ref[...], preferred_element_type=jnp.float32)
        + b_ref[...]
    )


def _outproj(x, w, b):
    return pl.pallas_call(
        _outproj_body,
        out_shape=jax.ShapeDtypeStruct((_BQ, _D), jnp.float32),
    )(x, w, b)


def kernel(query, reference_points, value, W_val, b_val, W_off, b_off,
           W_attn, b_attn, W_out, b_out):
    table = _vproj(value.reshape(_VROWS, _D), W_val, b_val[None, :])

    idxs, ws = _meta(
        query.reshape(_BQ, _D),
        W_off[:, 0::2], W_off[:, 1::2], W_attn,
        b_off[None, 0::2], b_off[None, 1::2], b_attn[None, :],
        jnp.asarray(_G_seg), jnp.asarray(_Ecat),
        reference_points.reshape(_BQ, 12),
        jnp.asarray(_SxA), jnp.asarray(_SyA),
    )

    sc_out = _sc_gather(table.reshape(_VROWS * _H, _HD),
                        idxs, ws.reshape(_BQ * 384))

    out = _outproj(sc_out.reshape(_BQ, _D), W_out, b_out[None, :])
    return out.reshape(_B, _Q, _D)
